# speedup vs baseline: 1.0435x; 1.0435x over previous
import functools

import jax
import jax.numpy as jnp
from jax import lax
from jax.experimental import pallas as pl
from jax.experimental.pallas import tpu as pltpu
from jax.experimental.pallas import tpu_sc as plsc

B = 16384
F = 26
FIELD_SIZE = 1000000
TABLE_ROWS = F * FIELD_SIZE
NC, NS, L = 2, 16, 16
NW = NC * NS
BPW = B // NW
CHUNK = 128
VPC = CHUNK // L
CPF = BPW // CHUNK
NCHUNK = F * BPW // CHUNK


def _body(xT, table, bias16, out, xv, idx_v, rows_v, out_v, bias_v, sem):
    wid = lax.axis_index("s") * NC + lax.axis_index("c")
    base = wid * BPW

    pltpu.sync_copy(xT.at[:, pl.ds(base, BPW)], xv)
    pltpu.sync_copy(bias16, bias_v)

    def fire(g, _):
        f = g // CPF
        part = g - f * CPF
        off = jnp.full((L,), f * FIELD_SIZE, jnp.int32)
        for j in range(VPC):
            idx_v[g, pl.ds(j * L, L)] = (
                xv[f, pl.ds(part * CHUNK + j * L, L)] + off
            )
        pltpu.make_async_copy(
            table.at[idx_v.at[g]], rows_v.at[pl.ds(g * CHUNK, CHUNK)], sem
        ).start()
        return 0

    lax.fori_loop(0, NCHUNK, fire, 0)

    def drain(g, _):
        pltpu.make_async_copy(
            table.at[idx_v.at[0]], rows_v.at[pl.ds(0, CHUNK)], sem
        ).wait()
        return 0

    lax.fori_loop(0, NCHUNK, drain, 0)

    def reduce_col(c, _):
        def inner(f, acc):
            return acc + rows_v[pl.ds(f * BPW + c * L, L)]

        acc = lax.fori_loop(0, F, inner, bias_v[...])
        out_v[pl.ds(c * L, L)] = 1.0 / (1.0 + jnp.exp(-acc))
        return 0

    lax.fori_loop(0, BPW // L, reduce_col, 0)

    pltpu.sync_copy(out_v, out.at[pl.ds(base, BPW)])


@functools.partial(
    pl.kernel,
    out_type=jax.ShapeDtypeStruct((B,), jnp.float32),
    mesh=plsc.VectorSubcoreMesh(core_axis_name="c", subcore_axis_name="s"),
    scratch_types=[
        pltpu.VMEM((F, BPW), jnp.int32),
        pltpu.VMEM((NCHUNK, CHUNK), jnp.int32),
        pltpu.VMEM((F * BPW,), jnp.float32),
        pltpu.VMEM((BPW,), jnp.float32),
        pltpu.VMEM((L,), jnp.float32),
        pltpu.SemaphoreType.DMA,
    ],
)
def _lr_kernel(xT, table, bias16, out, xv, idx_v, rows_v, out_v, bias_v, sem):
    _body(xT, table, bias16, out, xv, idx_v, rows_v, out_v, bias_v, sem)


def kernel(x, table, bias):
    xT = x.T
    table_flat = table.reshape(TABLE_ROWS)
    bias16 = jnp.broadcast_to(bias.astype(jnp.float32), (L,))
    return _lr_kernel(xT, table_flat, bias16)
